# Initial kernel scaffold; baseline (speedup 1.0000x reference)
import jax
import jax.numpy as jnp
from jax.experimental import pallas as pl


def kernel(x, edge_index, batch, W1, b1, g1, be1, W2, b2, g2, be2, Wh1, bh1, Wh2, bh2):
    raise NotImplementedError("write your pallas kernel here")


# speedup vs baseline: 13.0414x; 13.0414x over previous
import jax
import jax.numpy as jnp
from jax import lax
from jax.experimental import pallas as pl
from jax.experimental.pallas import tpu as pltpu
from jax.experimental.pallas import tpu_sc as plsc

N = 10000
E = 320000
D = 128
G = 64
NC, NS = 2, 16
NW = NC * NS
CH = 128
NCH = -(-E // (NW * CH))
EPT = NCH * CH
EPAD = NW * EPT
ACC = 10240
RPT = ACC // NS
TRASH = N
BLK = 1000
NBLK = N // BLK
BN_S = 1.0 / (1.0 + 1e-5) ** 0.5

def _hist_body(dst_hbm, ones_hbm, zeros_hbm, out_hbm, dstv, onesv, acc):
    c = lax.axis_index("c")
    s = lax.axis_index("s")
    wid = c * NS + s
    pltpu.sync_copy(zeros_hbm, acc.at[pl.ds(s * RPT, RPT)])
    pltpu.sync_copy(ones_hbm, onesv)
    pltpu.sync_copy(dst_hbm.at[wid], dstv)
    plsc.subcore_barrier()

    def body(j, carry):
        pltpu.sync_copy(onesv, acc.at[dstv.at[j]], add=True)
        return carry

    lax.fori_loop(0, NCH, body, 0)
    plsc.subcore_barrier()
    pltpu.sync_copy(acc.at[pl.ds(s * RPT, RPT)],
                    out_hbm.at[c, pl.ds(s * RPT, RPT)])


_sc_cache = {}


def _sc_kernels():
    if "hist" not in _sc_cache:
        mesh = plsc.VectorSubcoreMesh(
            core_axis_name="c", subcore_axis_name="s",
            num_cores=NC, num_subcores=NS)
        _sc_cache["hist"] = pl.kernel(
            _hist_body,
            out_type=jax.ShapeDtypeStruct((NC, ACC), jnp.float32),
            mesh=mesh,
            scratch_types=[
                pltpu.VMEM((NCH, CH), jnp.int32),
                pltpu.VMEM((CH,), jnp.float32),
                pltpu.VMEM_SHARED((ACC,), jnp.float32),
            ],
        )
        _sc_cache["msg"] = pl.kernel(
            _msg_body,
            out_type=jax.ShapeDtypeStruct((NC, ACC, D), jnp.float32),
            mesh=mesh,
            scratch_types=[
                pltpu.VMEM((NCH, CH), jnp.int32),
                pltpu.VMEM((NCH, CH), jnp.int32),
                pltpu.VMEM((CH, D), jnp.float32),
                pltpu.VMEM_SHARED((ACC, D), jnp.float32),
                pltpu.SemaphoreType.DMA,
            ],
        )
    return _sc_cache["hist"], _sc_cache["msg"]


def _msg_body(xs_hbm, src_hbm, dst_hbm, zeros_hbm, out_hbm,
              srcv, dstv, rows, acc, sem):
    c = lax.axis_index("c")
    s = lax.axis_index("s")
    wid = c * NS + s
    pltpu.sync_copy(zeros_hbm, acc.at[pl.ds(s * RPT, RPT)])
    pltpu.sync_copy(src_hbm.at[wid], srcv)
    pltpu.sync_copy(dst_hbm.at[wid], dstv)
    plsc.subcore_barrier()

    def body(j, carry):
        pltpu.async_copy(xs_hbm.at[srcv.at[j]], rows, sem).wait()
        pltpu.sync_copy(rows, acc.at[dstv.at[j]], add=True)
        return carry

    lax.fori_loop(0, NCH, body, 0)
    plsc.subcore_barrier()
    pltpu.sync_copy(acc.at[pl.ds(s * RPT, RPT)],
                    out_hbm.at[c, pl.ds(s * RPT, RPT)])


def _k1_body(x_ref, w_ref, p0_ref, p1_ref, o_ref):
    dinv = lax.rsqrt(1.0 + p0_ref[:, :1] + p1_ref[:, :1])
    o_ref[...] = jnp.dot(x_ref[...], w_ref[...],
                         preferred_element_type=jnp.float32) * dinv


def _k2_body(a0, a1, xs, p0, p1, b1, g1, be1, w2, o):
    dinv = lax.rsqrt(1.0 + p0[:, :1] + p1[:, :1])
    pre = dinv * (a0[...] + a1[...] + xs[...]) + b1[...]
    h = jnp.maximum(pre * (g1[...] * BN_S) + be1[...], 0.0)
    o[...] = jnp.dot(h, w2[...], preferred_element_type=jnp.float32) * dinv


def _k3_body(a0, a1, xs, p0, p1, b2, g2, be2, bt, wh1, bh1, wh2, bh2,
             o, sums, counts):
    i = pl.program_id(0)

    @pl.when(i == 0)
    def _():
        sums[...] = jnp.zeros_like(sums)
        counts[...] = jnp.zeros_like(counts)

    dinv = lax.rsqrt(1.0 + p0[:, :1] + p1[:, :1])
    pre = dinv * (a0[...] + a1[...] + xs[...]) + b2[...]
    h = jnp.maximum(pre * (g2[...] * BN_S) + be2[...], 0.0)
    b = bt[...].reshape(1, BLK)
    onehot = (lax.broadcasted_iota(jnp.int32, (G, BLK), 0)
              == jnp.broadcast_to(b, (G, BLK))).astype(jnp.float32)
    sums[...] += jnp.dot(onehot, h, preferred_element_type=jnp.float32)
    counts[...] += jnp.broadcast_to(
        jnp.sum(onehot, axis=1, keepdims=True), (G, D))

    @pl.when(i == NBLK - 1)
    def _():
        pooled = sums[...] / jnp.maximum(counts[...], 1.0)
        hh = jnp.maximum(
            jnp.dot(pooled, wh1[...], preferred_element_type=jnp.float32)
            + bh1[...], 0.0)
        o[...] = jnp.dot(hh, wh2[...],
                         preferred_element_type=jnp.float32) + bh2[...]


def _row_spec(i):
    return (i, 0)


def _fix_spec(i):
    return (0, 0)


def kernel(x, edge_index, batch, W1, b1, g1, be1, W2, b2, g2, be2,
           Wh1, bh1, Wh2, bh2):
    src = edge_index[0].astype(jnp.int32)
    dst = edge_index[1].astype(jnp.int32)
    pad = EPAD - E
    src_p = jnp.concatenate(
        [src, jnp.zeros((pad,), jnp.int32)]).reshape(NW, NCH, CH)
    dst_p = jnp.concatenate(
        [dst, jnp.full((pad,), TRASH, jnp.int32)]).reshape(NW, NCH, CH)
    ones_dw = jnp.ones((CH,), jnp.float32)
    zeros_dw = jnp.zeros((RPT,), jnp.float32)
    zeros_d = jnp.zeros((RPT, D), jnp.float32)

    _hist, _msg = _sc_kernels()
    degp = _hist(dst_p, ones_dw, zeros_dw)
    p0 = degp[0, :N].reshape(N, 1)
    p1 = degp[1, :N].reshape(N, 1)

    blk_d = pl.BlockSpec((BLK, D), _row_spec)
    blk_dw = pl.BlockSpec((BLK, 1), _row_spec)
    full_dd = pl.BlockSpec((D, D), _fix_spec)
    vec_d = pl.BlockSpec((1, D), _fix_spec)

    xs = pl.pallas_call(
        _k1_body, grid=(NBLK,),
        in_specs=[blk_d, full_dd, blk_dw, blk_dw],
        out_specs=blk_d,
        out_shape=jax.ShapeDtypeStruct((N, D), jnp.float32),
    )(x, W1, p0, p1)

    acc = _msg(xs, src_p, dst_p, zeros_d)

    xs2 = pl.pallas_call(
        _k2_body, grid=(NBLK,),
        in_specs=[blk_d, blk_d, blk_d, blk_dw, blk_dw,
                  vec_d, vec_d, vec_d, full_dd],
        out_specs=blk_d,
        out_shape=jax.ShapeDtypeStruct((N, D), jnp.float32),
    )(acc[0, :N], acc[1, :N], xs, p0, p1,
      b1.reshape(1, D), g1.reshape(1, D), be1.reshape(1, D), W2)

    acc2 = _msg(xs2, src_p, dst_p, zeros_d)

    batch_p = batch.astype(jnp.int32).reshape(NBLK, 1, BLK)
    Wh2p = jnp.pad(Wh2, ((0, 0), (0, D - Wh2.shape[1])))
    bh2p = jnp.pad(bh2, (0, D - bh2.shape[0])).reshape(1, D)

    out = pl.pallas_call(
        _k3_body, grid=(NBLK,),
        in_specs=[blk_d, blk_d, blk_d, blk_dw, blk_dw,
                  vec_d, vec_d, vec_d,
                  pl.BlockSpec((1, 1, BLK), lambda i: (i, 0, 0)),
                  full_dd, vec_d, full_dd, vec_d],
        out_specs=pl.BlockSpec((G, D), _fix_spec),
        out_shape=jax.ShapeDtypeStruct((G, D), jnp.float32),
        scratch_shapes=[pltpu.VMEM((G, D), jnp.float32),
                        pltpu.VMEM((G, D), jnp.float32)],
    )(acc2[0, :N], acc2[1, :N], xs2, p0, p1,
      b2.reshape(1, D), g2.reshape(1, D), be2.reshape(1, D),
      batch_p, Wh1, bh1.reshape(1, D), Wh2p, bh2p)

    return out[:, :Wh2.shape[1]]
